# baseline (speedup 1.0000x reference)
import dataclasses
import functools

import jax
import jax.numpy as jnp
from jax import lax
from jax.experimental import pallas as pl
from jax.experimental.pallas import tpu as pltpu
from jax.experimental.pallas import tpu_sc as plsc

_CHUNK = 128
_BN = 1024
_BE = 2048
_NW = 32
_L = 16


def _silu(x):
    return x * jax.nn.sigmoid(x)


def _sc_params():
    cp = pltpu.CompilerParams()
    if "needs_layout_passes" in pltpu.CompilerParams.__dataclass_fields__:
        cp = dataclasses.replace(cp, needs_layout_passes=False)
    return cp


def _chunk_to_col(x16, be):
    rows = lax.broadcasted_iota(jnp.int32, (be, 128), 0)
    lanes = lax.broadcasted_iota(jnp.int32, (be, 128), 1)
    lmask = (lanes == (rows % 128)).astype(jnp.float32)
    r0 = lax.broadcasted_iota(jnp.int32, (be, be // 128), 0)
    r1 = lax.broadcasted_iota(jnp.int32, (be, be // 128), 1)
    sel = ((r0 // 128) == r1).astype(jnp.float32)
    v = jnp.dot(sel, x16, preferred_element_type=jnp.float32)
    return jnp.sum(v * lmask, axis=1, keepdims=True)


def _col_to_chunk(vcol, be):
    rows = lax.broadcasted_iota(jnp.int32, (be, 128), 0)
    lanes = lax.broadcasted_iota(jnp.int32, (be, 128), 1)
    lmask = (lanes == (rows % 128)).astype(jnp.float32)
    c0 = lax.broadcasted_iota(jnp.int32, (be // 128, be), 0)
    c1 = lax.broadcasted_iota(jnp.int32, (be // 128, be), 1)
    selt = ((c1 // 128) == c0).astype(jnp.float32)
    return jnp.dot(selt, vcol * lmask, preferred_element_type=jnp.float32)


def _pre_body(h_ref, w1a_ref, w1b_ref, ts_ref, td_ref):
    h = h_ref[...]
    ts_ref[...] = jnp.dot(h, w1a_ref[...], preferred_element_type=jnp.float32)
    td_ref[...] = jnp.dot(h, w1b_ref[...], preferred_element_type=jnp.float32)


def _pre_call(h_pad, w1a, w1b, npad):
    return pl.pallas_call(
        _pre_body,
        grid=(npad // _BN,),
        in_specs=[
            pl.BlockSpec((_BN, 128), lambda i: (i, 0)),
            pl.BlockSpec((128, 128), lambda i: (0, 0)),
            pl.BlockSpec((128, 128), lambda i: (0, 0)),
        ],
        out_specs=[
            pl.BlockSpec((_BN, 128), lambda i: (i, 0)),
            pl.BlockSpec((_BN, 128), lambda i: (i, 0)),
        ],
        out_shape=[
            jax.ShapeDtypeStruct((npad, 128), jnp.float32),
            jax.ShapeDtypeStruct((npad, 128), jnp.float32),
        ],
    )(h_pad, w1a, w1b)


_GFRAC0 = 1.00


def _gather_call(ts, td, posx, posy, src2d, dst2d, n_chunks, npad):
    epad = n_chunks * _CHUNK
    w0 = int(n_chunks * _GFRAC0 / 128) * 128
    cpw0 = w0 // 16
    cpw1 = (n_chunks - w0) // 16
    cpw = max(cpw0, cpw1)
    mesh = plsc.VectorSubcoreMesh(core_axis_name="c", subcore_axis_name="s")

    @functools.partial(
        pl.kernel,
        out_type=[
            jax.ShapeDtypeStruct((epad, 128), jnp.float32),
            jax.ShapeDtypeStruct((n_chunks, _CHUNK), jnp.float32),
            jax.ShapeDtypeStruct((n_chunks, _CHUNK), jnp.float32),
        ],
        mesh=mesh,
        scratch_types=[
            pltpu.VMEM((npad,), jnp.float32),
            pltpu.VMEM((npad,), jnp.float32),
            pltpu.VMEM((cpw, _CHUNK), jnp.int32),
            pltpu.VMEM((cpw, _CHUNK), jnp.int32),
            pltpu.VMEM((_CHUNK, 128), jnp.float32),
            pltpu.VMEM((_CHUNK, 128), jnp.float32),
            pltpu.VMEM((1, _CHUNK), jnp.float32),
            pltpu.VMEM((1, _CHUNK), jnp.float32),
            pltpu.VMEM((1, _CHUNK), jnp.float32),
            pltpu.VMEM((1, _CHUNK), jnp.float32),
            pltpu.SemaphoreType.DMA,
            pltpu.SemaphoreType.DMA,
            pltpu.SemaphoreType.DMA,
            pltpu.SemaphoreType.DMA,
        ],
        compiler_params=_sc_params(),
    )
    def gather_k(ts_hbm, td_hbm, px_hbm, py_hbm, si_hbm, di_hbm,
                 g_hbm, dx_hbm, dy_hbm, px_v, py_v, idx_s, idx_d,
                 bufa, bufb, dxa, dya, dxb, dyb,
                 semga, semgb, semwa, semwb):
        c = lax.axis_index("c")
        s = lax.axis_index("s")
        my_cpw = jnp.where(c == 0, cpw0, cpw1)
        base = jnp.where(c == 0, s * cpw0, w0 + s * cpw1)
        pltpu.sync_copy(px_hbm, px_v)
        pltpu.sync_copy(py_hbm, py_v)

        if cpw0 > 0:
            @pl.when(c == 0)
            def _():
                pltpu.sync_copy(si_hbm.at[pl.ds(s * cpw0, cpw0)],
                                idx_s.at[pl.ds(0, cpw0)])
                pltpu.sync_copy(di_hbm.at[pl.ds(s * cpw0, cpw0)],
                                idx_d.at[pl.ds(0, cpw0)])

        if cpw1 > 0:
            @pl.when(c == 1)
            def _():
                pltpu.sync_copy(si_hbm.at[pl.ds(w0 + s * cpw1, cpw1)],
                                idx_s.at[pl.ds(0, cpw1)])
                pltpu.sync_copy(di_hbm.at[pl.ds(w0 + s * cpw1, cpw1)],
                                idx_d.at[pl.ds(0, cpw1)])

        def g1_start(jj, buf, sem):
            pltpu.async_copy(ts_hbm.at[idx_s.at[jj]], buf, sem)

        def g1_wait(jj, buf, sem):
            pltpu.make_async_copy(ts_hbm.at[idx_s.at[jj]], buf, sem).wait()

        def g2_start(jj, buf, sem):
            pltpu.async_copy(td_hbm.at[idx_d.at[jj]], buf, sem, add=True)

        def g2_wait(jj, buf, sem):
            pltpu.make_async_copy(td_hbm.at[idx_d.at[jj]], buf, sem).wait()

        def poscalc(jj, dxv, dyv):
            for k in range(_CHUNK // _L):
                sl = pl.ds(k * _L, _L)
                i_s = idx_s[jj, sl]
                i_d = idx_d[jj, sl]
                dxv[0, sl] = (plsc.load_gather(px_v, [i_s])
                              - plsc.load_gather(px_v, [i_d]))
                dyv[0, sl] = (plsc.load_gather(py_v, [i_s])
                              - plsc.load_gather(py_v, [i_d]))

        def w_start(jj, buf, dxv, dyv, sem):
            ch = base + jj
            pltpu.async_copy(buf, g_hbm.at[pl.ds(ch * _CHUNK, _CHUNK)], sem)
            pltpu.async_copy(dxv, dx_hbm.at[pl.ds(ch, 1)], sem)
            pltpu.async_copy(dyv, dy_hbm.at[pl.ds(ch, 1)], sem)

        def w_wait(jj, buf, dxv, dyv, sem):
            ch = base + jj
            pltpu.make_async_copy(
                buf, g_hbm.at[pl.ds(ch * _CHUNK, _CHUNK)], sem).wait()
            pltpu.make_async_copy(dxv, dx_hbm.at[pl.ds(ch, 1)], sem).wait()
            pltpu.make_async_copy(dyv, dy_hbm.at[pl.ds(ch, 1)], sem).wait()

        @pl.when(my_cpw > 0)
        def _():
            g1_start(0, bufa, semga)

        @pl.loop(0, my_cpw, step=2)
        def _(jj):
            g1_wait(jj, bufa, semga)
            g2_start(jj, bufa, semga)

            @pl.when(jj > 0)
            def _():
                w_wait(jj - 1, bufb, dxb, dyb, semwb)

            g1_start(jj + 1, bufb, semgb)
            poscalc(jj, dxa, dya)
            g2_wait(jj, bufa, semga)
            w_start(jj, bufa, dxa, dya, semwa)
            g1_wait(jj + 1, bufb, semgb)
            g2_start(jj + 1, bufb, semgb)
            poscalc(jj + 1, dxb, dyb)
            w_wait(jj, bufa, dxa, dya, semwa)

            @pl.when(jj + 2 < my_cpw)
            def _():
                g1_start(jj + 2, bufa, semga)

            g2_wait(jj + 1, bufb, semgb)
            w_start(jj + 1, bufb, dxb, dyb, semwb)

        @pl.when(my_cpw > 0)
        def _():
            w_wait(my_cpw - 1, bufb, dxb, dyb, semwb)

    return gather_k(ts, td, posx, posy, src2d, dst2d)


def _edge_body(g_ref, dx_ref, dy_ref, lmask_ref, sel_ref, selt_ref,
               w2t_ref, w1c_ref, be1_ref, be2_ref, wc_ref, bc_ref,
               mo_ref, wxy_ref):
    dx16 = dx_ref[...]
    dy16 = dy_ref[...]
    lmask = lmask_ref[...]
    dsq16 = dx16 * dx16 + dy16 * dy16
    dsq = jnp.sum(
        jnp.dot(sel_ref[...], dsq16, preferred_element_type=jnp.float32)
        * lmask, axis=1, keepdims=True)
    pre1 = g_ref[...] + dsq * w1c_ref[...] + be1_ref[...]
    m = _silu(pre1)
    m2 = jnp.dot(m.astype(jnp.bfloat16), w2t_ref[...],
                 preferred_element_type=jnp.float32)
    mij = _silu(m2 + be2_ref[...])
    tw = jnp.tanh(jnp.sum(mij * wc_ref[...], axis=1, keepdims=True)
                  + bc_ref[0, 0])
    tw16 = jnp.dot(selt_ref[...], tw * lmask,
                   preferred_element_type=jnp.float32)
    mo_ref[...] = mij
    wxy_ref[...] = jnp.concatenate([dx16 * tw16, dy16 * tw16], axis=1)


def _edge_call(gsum, dx2d, dy2d, lmask, sel, selt, w2t, w1c, be1, be2,
               wc, bc, epad):
    nc = _BE // 128
    full = lambda i: (0, 0)
    return pl.pallas_call(
        _edge_body,
        grid=(epad // _BE,),
        in_specs=[
            pl.BlockSpec((_BE, 128), lambda i: (i, 0)),
            pl.BlockSpec((nc, _CHUNK), lambda i: (i, 0)),
            pl.BlockSpec((nc, _CHUNK), lambda i: (i, 0)),
            pl.BlockSpec((_BE, 128), full),
            pl.BlockSpec((_BE, nc), full),
            pl.BlockSpec((nc, _BE), full),
            pl.BlockSpec((128, 128), full),
            pl.BlockSpec((1, 128), full),
            pl.BlockSpec((1, 128), full),
            pl.BlockSpec((1, 128), full),
            pl.BlockSpec((1, 128), full),
            pl.BlockSpec((1, 1), full),
        ],
        out_specs=[
            pl.BlockSpec((_BE, 128), lambda i: (i, 0)),
            pl.BlockSpec((nc, 2 * _CHUNK), lambda i: (i, 0)),
        ],
        out_shape=[
            jax.ShapeDtypeStruct((epad, 128), jnp.float32),
            jax.ShapeDtypeStruct((epad // 128, 2 * _CHUNK), jnp.float32),
        ],
    )(gsum, dx2d, dy2d, lmask, sel, selt, w2t, w1c, be1, be2, wc, bc)


def _scatter_call(dst2d, mout, wxy2d, zeros_hbm, n_chunks, npad):
    cpt = n_chunks // 16
    half = npad // 2
    accn = half + _CHUNK
    rows_per_tile = accn // 16
    nr = npad // 128
    mesh = plsc.VectorSubcoreMesh(core_axis_name="c", subcore_axis_name="s")

    @functools.partial(
        pl.kernel,
        out_type=[
            jax.ShapeDtypeStruct((2, accn, 128), jnp.float32),
            jax.ShapeDtypeStruct((_NW, nr, 128), jnp.float32),
        ],
        mesh=mesh,
        scratch_types=[
            pltpu.VMEM((cpt, _CHUNK), jnp.int32),
            pltpu.VMEM((1, _CHUNK), jnp.int32),
            pltpu.VMEM((_CHUNK, 128), jnp.float32),
            pltpu.VMEM((_CHUNK, 128), jnp.float32),
            pltpu.VMEM((1, 2 * _CHUNK), jnp.float32),
            pltpu.VMEM((1, 2 * _CHUNK), jnp.float32),
            pltpu.VMEM((nr, 128), jnp.float32),
            pltpu.VMEM_SHARED((accn, 128), jnp.float32),
            pltpu.SemaphoreType.DMA,
            pltpu.SemaphoreType.DMA,
            pltpu.SemaphoreType.DMA,
            pltpu.SemaphoreType.DMA,
        ],
        compiler_params=_sc_params(),
    )
    def scatter_k(di_hbm, mo_hbm, wxy_hbm, z_hbm, agg_hbm, av_hbm,
                  idx_v, idxm, buf0, buf1, wbuf0, wbuf1, accv, acc,
                  sem0, sem1, semw0, semw1):
        c = lax.axis_index("c")
        s = lax.axis_index("s")
        wid = c * 16 + s
        base = s * cpt
        nbase = c * half
        pltpu.sync_copy(z_hbm.at[pl.ds(s * rows_per_tile, rows_per_tile)],
                        acc.at[pl.ds(s * rows_per_tile, rows_per_tile)])
        pltpu.sync_copy(z_hbm.at[pl.ds(0, nr)], accv)
        pltpu.sync_copy(di_hbm.at[pl.ds(base, cpt)], idx_v)
        plsc.subcore_barrier()

        def start(ch, buf, wbuf, semr, semrw):
            pltpu.async_copy(
                mo_hbm.at[pl.ds(ch * _CHUNK, _CHUNK)], buf, semr)
            pltpu.async_copy(wxy_hbm.at[pl.ds(ch, 1)], wbuf, semrw)

        def wait(ch, buf, wbuf, semr, semrw):
            pltpu.make_async_copy(
                mo_hbm.at[pl.ds(ch * _CHUNK, _CHUNK)], buf, semr).wait()
            pltpu.make_async_copy(
                wxy_hbm.at[pl.ds(ch, 1)], wbuf, semrw).wait()

        def accum(jj, buf, wbuf):
            for k in range(_CHUNK // _L):
                sl = pl.ds(k * _L, _L)
                idx16 = idx_v[jj, sl]
                rel = idx16 - nbase
                inr = (rel >= 0) & (rel < half)
                idxm[0, sl] = jnp.where(inr, rel, half)
            pltpu.sync_copy(buf, acc.at[idxm.at[0]], add=True)
            for k in range(_CHUNK // _L):
                sl = pl.ds(k * _L, _L)
                idx16 = idx_v[jj, sl]
                r16 = lax.shift_right_logical(idx16, 7)
                c16 = lax.bitwise_and(idx16, 127)
                w16 = jnp.where(c == 0, wbuf[0, sl],
                                wbuf[0, pl.ds(_CHUNK + k * _L, _L)])
                plsc.addupdate_scatter(accv, [r16, c16], w16)

        start(base, buf0, wbuf0, sem0, semw0)

        @pl.loop(0, cpt, step=2)
        def _(jj):
            ch = base + jj
            wait(ch, buf0, wbuf0, sem0, semw0)
            start(ch + 1, buf1, wbuf1, sem1, semw1)
            accum(jj, buf0, wbuf0)
            wait(ch + 1, buf1, wbuf1, sem1, semw1)

            @pl.when(jj + 2 < cpt)
            def _():
                start(ch + 2, buf0, wbuf0, sem0, semw0)

            accum(jj + 1, buf1, wbuf1)

        plsc.subcore_barrier()
        pltpu.sync_copy(
            acc.at[pl.ds(s * rows_per_tile, rows_per_tile)],
            agg_hbm.at[c, pl.ds(s * rows_per_tile, rows_per_tile)])
        pltpu.sync_copy(accv, av_hbm.at[wid])

    return scatter_k(dst2d, mout, wxy2d, zeros_hbm)


def _node_body(h_ref, hv_ref, agg_ref, av_ref, wn1a_ref, wn1b_ref,
               bn1_ref, wn2_ref, bn2_ref, wg_ref, bg_ref, oh_ref, ov_ref):
    h = h_ref[...]
    aggs = agg_ref[0]
    av = av_ref[...]
    avx = _chunk_to_col(jnp.sum(av[:16], axis=0), _BN)
    avy = _chunk_to_col(jnp.sum(av[16:], axis=0), _BN)
    hn = _silu(jnp.dot(h, wn1a_ref[...], preferred_element_type=jnp.float32)
               + jnp.dot(aggs, wn1b_ref[...],
                         preferred_element_type=jnp.float32)
               + bn1_ref[...])
    hsn = jnp.dot(hn, wn2_ref[...], preferred_element_type=jnp.float32) \
        + bn2_ref[...]
    gate = jax.nn.sigmoid(jnp.sum(hsn * wg_ref[...], axis=1, keepdims=True)
                          + bg_ref[0, 0])
    oh_ref[...] = h + hsn
    ov_ref[...] = hv_ref[...] * gate + jnp.concatenate([avx, avy], axis=1)


def _node_call(h_pad, hv_pad, agg, av, wn1a, wn1b, bn1, wn2, bn2,
               wg, bg, npad):
    nc = _BN // 128
    bpc = (npad // 2) // _BN
    full = lambda i: (0, 0)
    return pl.pallas_call(
        _node_body,
        grid=(npad // _BN,),
        in_specs=[
            pl.BlockSpec((_BN, 128), lambda i: (i, 0)),
            pl.BlockSpec((_BN, 2), lambda i: (i, 0)),
            pl.BlockSpec((1, _BN, 128), lambda i: (i // bpc, i % bpc, 0)),
            pl.BlockSpec((_NW, nc, 128), lambda i: (0, i, 0)),
            pl.BlockSpec((128, 128), full),
            pl.BlockSpec((128, 128), full),
            pl.BlockSpec((1, 128), full),
            pl.BlockSpec((128, 128), full),
            pl.BlockSpec((1, 128), full),
            pl.BlockSpec((1, 128), full),
            pl.BlockSpec((1, 1), full),
        ],
        out_specs=[
            pl.BlockSpec((_BN, 128), lambda i: (i, 0)),
            pl.BlockSpec((_BN, 2), lambda i: (i, 0)),
        ],
        out_shape=[
            jax.ShapeDtypeStruct((npad, 128), jnp.float32),
            jax.ShapeDtypeStruct((npad, 2), jnp.float32),
        ],
    )(h_pad, hv_pad, agg, av, wn1a, wn1b, bn1, wn2, bn2, wg, bg)


def kernel(h_scalar, h_vector, pos, edge_index, W_e1, b_e1, W_e2, b_e2,
           W_n1, b_n1, W_n2, b_n2, W_c, b_c, W_g, b_g):
    n, sd = h_scalar.shape
    e = edge_index.shape[1]
    assert sd == 128

    npad = -(-n // (2 * _BN)) * (2 * _BN)
    n_chunks = -(-e // _CHUNK)
    cpw = -(-n_chunks // _NW)
    if cpw % 2:
        cpw += 1
    n_chunks = cpw * _NW
    epad = n_chunks * _CHUNK

    src = edge_index[0]
    dst = edge_index[1]
    src2d = jnp.concatenate(
        [src, jnp.zeros((epad - e,), jnp.int32)]).reshape(n_chunks, _CHUNK)
    dst2d = jnp.concatenate(
        [dst, jnp.full((epad - e,), n, jnp.int32)]).reshape(n_chunks, _CHUNK)

    h_pad = jnp.pad(h_scalar, ((0, npad - n), (0, 0)))
    hv_pad = jnp.pad(h_vector, ((0, npad - n), (0, 0)))
    pos_pad = jnp.pad(pos, ((0, npad - n), (0, 0)))
    posx = pos_pad[:, 0]
    posy = pos_pad[:, 1]

    w1a = W_e1[:, :sd].T
    w1b = W_e1[:, sd:2 * sd].T
    w1c = W_e1[:, 2 * sd].reshape(1, sd)
    w2t = W_e2.T
    wn1a = W_n1[:, :sd].T
    wn1b = W_n1[:, sd:].T
    wn2 = W_n2.T
    be1 = b_e1.reshape(1, sd)
    be2 = b_e2.reshape(1, sd)
    bn1 = b_n1.reshape(1, sd)
    bn2 = b_n2.reshape(1, sd)
    bc = b_c.reshape(1, 1)
    bg = b_g.reshape(1, 1)

    rows = jnp.arange(_BE, dtype=jnp.int32)
    lanes = jnp.arange(128, dtype=jnp.int32)
    lmask = (lanes[None, :] == (rows[:, None] % 128)).astype(jnp.float32)
    chk = jnp.arange(_BE // 128, dtype=jnp.int32)
    sel = ((rows[:, None] // 128) == chk[None, :]).astype(jnp.float32)
    selt = sel.T

    ts, td = _pre_call(h_pad, w1a, w1b, npad)
    gsum, dx2d, dy2d = _gather_call(ts, td, posx, posy, src2d, dst2d,
                                    n_chunks, npad)
    mout, wxy2d = _edge_call(gsum, dx2d, dy2d, lmask, sel, selt,
                             w2t.astype(jnp.bfloat16), w1c, be1, be2,
                             W_c, bc, epad)
    zeros_hbm = jnp.zeros((npad, 128), jnp.float32)
    agg, av = _scatter_call(dst2d, mout, wxy2d, zeros_hbm, n_chunks, npad)
    oh, ov = _node_call(h_pad, hv_pad, agg, av, wn1a, wn1b, bn1,
                        wn2, bn2, W_g, bg, npad)
    return (oh[:n], ov[:n])


# speedup vs baseline: 1.2920x; 1.2920x over previous
import dataclasses
import functools

import jax
import jax.numpy as jnp
from jax import lax
from jax.experimental import pallas as pl
from jax.experimental.pallas import tpu as pltpu
from jax.experimental.pallas import tpu_sc as plsc

_CHUNK = 128
_BN = 1024
_BE = 2048
_NW = 32
_L = 16


def _silu(x):
    return x * jax.nn.sigmoid(x)


def _sc_params():
    cp = pltpu.CompilerParams()
    if "needs_layout_passes" in pltpu.CompilerParams.__dataclass_fields__:
        cp = dataclasses.replace(cp, needs_layout_passes=False)
    return cp


def _chunk_to_col(x16, be):
    rows = lax.broadcasted_iota(jnp.int32, (be, 128), 0)
    lanes = lax.broadcasted_iota(jnp.int32, (be, 128), 1)
    lmask = (lanes == (rows % 128)).astype(jnp.float32)
    r0 = lax.broadcasted_iota(jnp.int32, (be, be // 128), 0)
    r1 = lax.broadcasted_iota(jnp.int32, (be, be // 128), 1)
    sel = ((r0 // 128) == r1).astype(jnp.float32)
    v = jnp.dot(sel, x16, preferred_element_type=jnp.float32)
    return jnp.sum(v * lmask, axis=1, keepdims=True)


def _col_to_chunk(vcol, be):
    rows = lax.broadcasted_iota(jnp.int32, (be, 128), 0)
    lanes = lax.broadcasted_iota(jnp.int32, (be, 128), 1)
    lmask = (lanes == (rows % 128)).astype(jnp.float32)
    c0 = lax.broadcasted_iota(jnp.int32, (be // 128, be), 0)
    c1 = lax.broadcasted_iota(jnp.int32, (be // 128, be), 1)
    selt = ((c1 // 128) == c0).astype(jnp.float32)
    return jnp.dot(selt, vcol * lmask, preferred_element_type=jnp.float32)


def _pre_body(h_ref, w1a_ref, w1b_ref, ts_ref, td_ref):
    h = h_ref[...]
    ts_ref[...] = jnp.dot(h, w1a_ref[...], preferred_element_type=jnp.float32)
    td_ref[...] = jnp.dot(h, w1b_ref[...], preferred_element_type=jnp.float32)


def _pre_call(h_pad, w1a, w1b, npad):
    return pl.pallas_call(
        _pre_body,
        grid=(npad // _BN,),
        in_specs=[
            pl.BlockSpec((_BN, 128), lambda i: (i, 0)),
            pl.BlockSpec((128, 128), lambda i: (0, 0)),
            pl.BlockSpec((128, 128), lambda i: (0, 0)),
        ],
        out_specs=[
            pl.BlockSpec((_BN, 128), lambda i: (i, 0)),
            pl.BlockSpec((_BN, 128), lambda i: (i, 0)),
        ],
        out_shape=[
            jax.ShapeDtypeStruct((npad, 128), jnp.float32),
            jax.ShapeDtypeStruct((npad, 128), jnp.float32),
        ],
    )(h_pad, w1a, w1b)


_GFRAC0 = 0.60
_NSEG = 4


def _gather_call(ts, td, posx, posy, src2d, dst2d, n_chunks, npad):
    epad = n_chunks * _CHUNK
    w0 = int(n_chunks * _GFRAC0 / 128) * 128
    cpw0 = w0 // 16
    cpw1 = (n_chunks - w0) // 16
    cpw = max(cpw0, cpw1)
    mesh = plsc.VectorSubcoreMesh(core_axis_name="c", subcore_axis_name="s")

    @functools.partial(
        pl.kernel,
        out_type=[
            jax.ShapeDtypeStruct((epad, 128), jnp.float32),
            jax.ShapeDtypeStruct((n_chunks, _CHUNK), jnp.float32),
            jax.ShapeDtypeStruct((n_chunks, _CHUNK), jnp.float32),
        ],
        mesh=mesh,
        scratch_types=[
            pltpu.VMEM((npad,), jnp.float32),
            pltpu.VMEM((npad,), jnp.float32),
            pltpu.VMEM((cpw, _CHUNK), jnp.int32),
            pltpu.VMEM((cpw, _CHUNK), jnp.int32),
            pltpu.VMEM((_CHUNK, 128), jnp.float32),
            pltpu.VMEM((_CHUNK, 128), jnp.float32),
            pltpu.VMEM((1, _CHUNK), jnp.float32),
            pltpu.VMEM((1, _CHUNK), jnp.float32),
            pltpu.VMEM((1, _CHUNK), jnp.float32),
            pltpu.VMEM((1, _CHUNK), jnp.float32),
            pltpu.SemaphoreType.DMA,
            pltpu.SemaphoreType.DMA,
            pltpu.SemaphoreType.DMA,
            pltpu.SemaphoreType.DMA,
        ],
        compiler_params=_sc_params(),
    )
    def gather_k(ts_hbm, td_hbm, px_hbm, py_hbm, si_hbm, di_hbm,
                 g_hbm, dx_hbm, dy_hbm, px_v, py_v, idx_s, idx_d,
                 bufa, bufb, dxa, dya, dxb, dyb,
                 semga, semgb, semwa, semwb):
        c = lax.axis_index("c")
        s = lax.axis_index("s")
        my_cpw = jnp.where(c == 0, cpw0, cpw1)
        base = jnp.where(c == 0, s * cpw0, w0 + s * cpw1)
        pltpu.sync_copy(px_hbm, px_v)
        pltpu.sync_copy(py_hbm, py_v)

        if cpw0 > 0:
            @pl.when(c == 0)
            def _():
                pltpu.sync_copy(si_hbm.at[pl.ds(s * cpw0, cpw0)],
                                idx_s.at[pl.ds(0, cpw0)])
                pltpu.sync_copy(di_hbm.at[pl.ds(s * cpw0, cpw0)],
                                idx_d.at[pl.ds(0, cpw0)])

        if cpw1 > 0:
            @pl.when(c == 1)
            def _():
                pltpu.sync_copy(si_hbm.at[pl.ds(w0 + s * cpw1, cpw1)],
                                idx_s.at[pl.ds(0, cpw1)])
                pltpu.sync_copy(di_hbm.at[pl.ds(w0 + s * cpw1, cpw1)],
                                idx_d.at[pl.ds(0, cpw1)])

        def g1_start(jj, buf, sem):
            pltpu.async_copy(ts_hbm.at[idx_s.at[jj]], buf, sem)

        def g1_wait(jj, buf, sem):
            pltpu.make_async_copy(ts_hbm.at[idx_s.at[jj]], buf, sem).wait()

        def g2_start(jj, buf, sem):
            pltpu.async_copy(td_hbm.at[idx_d.at[jj]], buf, sem, add=True)

        def g2_wait(jj, buf, sem):
            pltpu.make_async_copy(td_hbm.at[idx_d.at[jj]], buf, sem).wait()

        def poscalc(jj, dxv, dyv):
            for k in range(_CHUNK // _L):
                sl = pl.ds(k * _L, _L)
                i_s = idx_s[jj, sl]
                i_d = idx_d[jj, sl]
                dxv[0, sl] = (plsc.load_gather(px_v, [i_s])
                              - plsc.load_gather(px_v, [i_d]))
                dyv[0, sl] = (plsc.load_gather(py_v, [i_s])
                              - plsc.load_gather(py_v, [i_d]))

        def w_start(jj, buf, dxv, dyv, sem):
            ch = base + jj
            pltpu.async_copy(buf, g_hbm.at[pl.ds(ch * _CHUNK, _CHUNK)], sem)
            pltpu.async_copy(dxv, dx_hbm.at[pl.ds(ch, 1)], sem)
            pltpu.async_copy(dyv, dy_hbm.at[pl.ds(ch, 1)], sem)

        def w_wait(jj, buf, dxv, dyv, sem):
            ch = base + jj
            pltpu.make_async_copy(
                buf, g_hbm.at[pl.ds(ch * _CHUNK, _CHUNK)], sem).wait()
            pltpu.make_async_copy(dxv, dx_hbm.at[pl.ds(ch, 1)], sem).wait()
            pltpu.make_async_copy(dyv, dy_hbm.at[pl.ds(ch, 1)], sem).wait()

        @pl.when(my_cpw > 0)
        def _():
            g1_start(0, bufa, semga)

        @pl.loop(0, my_cpw, step=2)
        def _(jj):
            g1_wait(jj, bufa, semga)
            g2_start(jj, bufa, semga)

            @pl.when(jj > 0)
            def _():
                w_wait(jj - 1, bufb, dxb, dyb, semwb)

            g1_start(jj + 1, bufb, semgb)
            poscalc(jj, dxa, dya)
            g2_wait(jj, bufa, semga)
            w_start(jj, bufa, dxa, dya, semwa)
            g1_wait(jj + 1, bufb, semgb)
            g2_start(jj + 1, bufb, semgb)
            poscalc(jj + 1, dxb, dyb)
            w_wait(jj, bufa, dxa, dya, semwa)

            @pl.when(jj + 2 < my_cpw)
            def _():
                g1_start(jj + 2, bufa, semga)

            g2_wait(jj + 1, bufb, semgb)
            w_start(jj + 1, bufb, dxb, dyb, semwb)

        @pl.when(my_cpw > 0)
        def _():
            w_wait(my_cpw - 1, bufb, dxb, dyb, semwb)

    return gather_k(ts, td, posx, posy, src2d, dst2d)


def _edge_body(g_ref, dx_ref, dy_ref, lmask_ref, sel_ref, selt_ref,
               w2t_ref, w1c_ref, be1_ref, be2_ref, wc_ref, bc_ref,
               mo_ref, wxy_ref):
    dx16 = dx_ref[...]
    dy16 = dy_ref[...]
    lmask = lmask_ref[...]
    dsq16 = dx16 * dx16 + dy16 * dy16
    dsq = jnp.sum(
        jnp.dot(sel_ref[...], dsq16, preferred_element_type=jnp.float32)
        * lmask, axis=1, keepdims=True)
    pre1 = g_ref[...] + dsq * w1c_ref[...] + be1_ref[...]
    m = _silu(pre1)
    m2 = jnp.dot(m.astype(jnp.bfloat16), w2t_ref[...],
                 preferred_element_type=jnp.float32)
    mij = _silu(m2 + be2_ref[...])
    tw = jnp.tanh(jnp.sum(mij * wc_ref[...], axis=1, keepdims=True)
                  + bc_ref[0, 0])
    tw16 = jnp.dot(selt_ref[...], tw * lmask,
                   preferred_element_type=jnp.float32)
    mo_ref[...] = mij
    wxy_ref[...] = jnp.concatenate([dx16 * tw16, dy16 * tw16], axis=1)


def _edge_call(gsum, dx2d, dy2d, lmask, sel, selt, w2t, w1c, be1, be2,
               wc, bc, epad):
    nc = _BE // 128
    full = lambda i: (0, 0)
    return pl.pallas_call(
        _edge_body,
        grid=(epad // _BE,),
        in_specs=[
            pl.BlockSpec((_BE, 128), lambda i: (i, 0)),
            pl.BlockSpec((nc, _CHUNK), lambda i: (i, 0)),
            pl.BlockSpec((nc, _CHUNK), lambda i: (i, 0)),
            pl.BlockSpec((_BE, 128), full),
            pl.BlockSpec((_BE, nc), full),
            pl.BlockSpec((nc, _BE), full),
            pl.BlockSpec((128, 128), full),
            pl.BlockSpec((1, 128), full),
            pl.BlockSpec((1, 128), full),
            pl.BlockSpec((1, 128), full),
            pl.BlockSpec((1, 128), full),
            pl.BlockSpec((1, 1), full),
        ],
        out_specs=[
            pl.BlockSpec((_BE, 128), lambda i: (i, 0)),
            pl.BlockSpec((nc, 2 * _CHUNK), lambda i: (i, 0)),
        ],
        out_shape=[
            jax.ShapeDtypeStruct((epad, 128), jnp.float32),
            jax.ShapeDtypeStruct((epad // 128, 2 * _CHUNK), jnp.float32),
        ],
    )(gsum, dx2d, dy2d, lmask, sel, selt, w2t, w1c, be1, be2, wc, bc)


def _scatter_call(dst2d, mout, wxy2d, zeros_hbm, n_chunks, npad):
    cpt = n_chunks // 16
    half = npad // 2
    accn = half + _CHUNK
    rows_per_tile = accn // 16
    nr = npad // 128
    mesh = plsc.VectorSubcoreMesh(core_axis_name="c", subcore_axis_name="s")

    @functools.partial(
        pl.kernel,
        out_type=[
            jax.ShapeDtypeStruct((2, accn, 128), jnp.float32),
            jax.ShapeDtypeStruct((_NW, nr, 128), jnp.float32),
        ],
        mesh=mesh,
        scratch_types=[
            pltpu.VMEM((cpt, _CHUNK), jnp.int32),
            pltpu.VMEM((1, _CHUNK), jnp.int32),
            pltpu.VMEM((_CHUNK, 128), jnp.float32),
            pltpu.VMEM((_CHUNK, 128), jnp.float32),
            pltpu.VMEM((1, 2 * _CHUNK), jnp.float32),
            pltpu.VMEM((1, 2 * _CHUNK), jnp.float32),
            pltpu.VMEM((nr, 128), jnp.float32),
            pltpu.VMEM_SHARED((accn, 128), jnp.float32),
            pltpu.SemaphoreType.DMA,
            pltpu.SemaphoreType.DMA,
            pltpu.SemaphoreType.DMA,
            pltpu.SemaphoreType.DMA,
        ],
        compiler_params=_sc_params(),
    )
    def scatter_k(di_hbm, mo_hbm, wxy_hbm, z_hbm, agg_hbm, av_hbm,
                  idx_v, idxm, buf0, buf1, wbuf0, wbuf1, accv, acc,
                  sem0, sem1, semw0, semw1):
        c = lax.axis_index("c")
        s = lax.axis_index("s")
        wid = c * 16 + s
        base = s * cpt
        nbase = c * half
        pltpu.sync_copy(z_hbm.at[pl.ds(s * rows_per_tile, rows_per_tile)],
                        acc.at[pl.ds(s * rows_per_tile, rows_per_tile)])
        pltpu.sync_copy(z_hbm.at[pl.ds(0, nr)], accv)
        pltpu.sync_copy(di_hbm.at[pl.ds(base, cpt)], idx_v)
        plsc.subcore_barrier()

        def start(ch, buf, wbuf, semr, semrw):
            pltpu.async_copy(
                mo_hbm.at[pl.ds(ch * _CHUNK, _CHUNK)], buf, semr)
            pltpu.async_copy(wxy_hbm.at[pl.ds(ch, 1)], wbuf, semrw)

        def wait(ch, buf, wbuf, semr, semrw):
            pltpu.make_async_copy(
                mo_hbm.at[pl.ds(ch * _CHUNK, _CHUNK)], buf, semr).wait()
            pltpu.make_async_copy(
                wxy_hbm.at[pl.ds(ch, 1)], wbuf, semrw).wait()

        def accum(jj, buf, wbuf):
            for k in range(_CHUNK // _L):
                sl = pl.ds(k * _L, _L)
                idx16 = idx_v[jj, sl]
                rel = idx16 - nbase
                inr = (rel >= 0) & (rel < half)
                idxm[0, sl] = jnp.where(inr, rel, half)
            pltpu.sync_copy(buf, acc.at[idxm.at[0]], add=True)
            for k in range(_CHUNK // _L):
                sl = pl.ds(k * _L, _L)
                idx16 = idx_v[jj, sl]
                r16 = lax.shift_right_logical(idx16, 7)
                c16 = lax.bitwise_and(idx16, 127)
                w16 = jnp.where(c == 0, wbuf[0, sl],
                                wbuf[0, pl.ds(_CHUNK + k * _L, _L)])
                plsc.addupdate_scatter(accv, [r16, c16], w16)

        start(base, buf0, wbuf0, sem0, semw0)

        @pl.loop(0, cpt, step=2)
        def _(jj):
            ch = base + jj
            wait(ch, buf0, wbuf0, sem0, semw0)
            start(ch + 1, buf1, wbuf1, sem1, semw1)
            accum(jj, buf0, wbuf0)
            wait(ch + 1, buf1, wbuf1, sem1, semw1)

            @pl.when(jj + 2 < cpt)
            def _():
                start(ch + 2, buf0, wbuf0, sem0, semw0)

            accum(jj + 1, buf1, wbuf1)

        plsc.subcore_barrier()
        pltpu.sync_copy(
            acc.at[pl.ds(s * rows_per_tile, rows_per_tile)],
            agg_hbm.at[c, pl.ds(s * rows_per_tile, rows_per_tile)])
        pltpu.sync_copy(accv, av_hbm.at[wid])

    return scatter_k(dst2d, mout, wxy2d, zeros_hbm)


def _node_body(h_ref, hv_ref, *rest):
    agg_refs = rest[:_NSEG]
    av_refs = rest[_NSEG:2 * _NSEG]
    (wn1a_ref, wn1b_ref, bn1_ref, wn2_ref, bn2_ref, wg_ref, bg_ref,
     oh_ref, ov_ref) = rest[2 * _NSEG:]
    h = h_ref[...]
    aggs = agg_refs[0][0]
    for r in agg_refs[1:]:
        aggs = aggs + r[0]
    av = av_refs[0][...]
    for r in av_refs[1:]:
        av = av + r[...]
    avx = _chunk_to_col(jnp.sum(av[:16], axis=0), _BN)
    avy = _chunk_to_col(jnp.sum(av[16:], axis=0), _BN)
    hn = _silu(jnp.dot(h, wn1a_ref[...], preferred_element_type=jnp.float32)
               + jnp.dot(aggs, wn1b_ref[...],
                         preferred_element_type=jnp.float32)
               + bn1_ref[...])
    hsn = jnp.dot(hn, wn2_ref[...], preferred_element_type=jnp.float32) \
        + bn2_ref[...]
    gate = jax.nn.sigmoid(jnp.sum(hsn * wg_ref[...], axis=1, keepdims=True)
                          + bg_ref[0, 0])
    oh_ref[...] = h + hsn
    ov_ref[...] = hv_ref[...] * gate + jnp.concatenate([avx, avy], axis=1)


def _node_call(h_pad, hv_pad, aggs, avs, wn1a, wn1b, bn1, wn2, bn2,
               wg, bg, npad):
    nc = _BN // 128
    bpc = (npad // 2) // _BN
    full = lambda i: (0, 0)
    return pl.pallas_call(
        _node_body,
        grid=(npad // _BN,),
        in_specs=[
            pl.BlockSpec((_BN, 128), lambda i: (i, 0)),
            pl.BlockSpec((_BN, 2), lambda i: (i, 0)),
        ] + [
            pl.BlockSpec((1, _BN, 128), lambda i: (i // bpc, i % bpc, 0))
            for _ in range(_NSEG)
        ] + [
            pl.BlockSpec((_NW, nc, 128), lambda i: (0, i, 0))
            for _ in range(_NSEG)
        ] + [
            pl.BlockSpec((128, 128), full),
            pl.BlockSpec((128, 128), full),
            pl.BlockSpec((1, 128), full),
            pl.BlockSpec((128, 128), full),
            pl.BlockSpec((1, 128), full),
            pl.BlockSpec((1, 128), full),
            pl.BlockSpec((1, 1), full),
        ],
        out_specs=[
            pl.BlockSpec((_BN, 128), lambda i: (i, 0)),
            pl.BlockSpec((_BN, 2), lambda i: (i, 0)),
        ],
        out_shape=[
            jax.ShapeDtypeStruct((npad, 128), jnp.float32),
            jax.ShapeDtypeStruct((npad, 2), jnp.float32),
        ],
    )(h_pad, hv_pad, *aggs, *avs, wn1a, wn1b, bn1, wn2, bn2, wg, bg)


def kernel(h_scalar, h_vector, pos, edge_index, W_e1, b_e1, W_e2, b_e2,
           W_n1, b_n1, W_n2, b_n2, W_c, b_c, W_g, b_g):
    n, sd = h_scalar.shape
    e = edge_index.shape[1]
    assert sd == 128

    npad = -(-n // (2 * _BN)) * (2 * _BN)
    n_chunks = -(-e // _CHUNK)
    cpw = -(-n_chunks // _NW)
    if cpw % 2:
        cpw += 1
    n_chunks = cpw * _NW
    epad = n_chunks * _CHUNK

    src = edge_index[0]
    dst = edge_index[1]
    src2d = jnp.concatenate(
        [src, jnp.zeros((epad - e,), jnp.int32)]).reshape(n_chunks, _CHUNK)
    dst2d = jnp.concatenate(
        [dst, jnp.full((epad - e,), n, jnp.int32)]).reshape(n_chunks, _CHUNK)

    h_pad = jnp.pad(h_scalar, ((0, npad - n), (0, 0)))
    hv_pad = jnp.pad(h_vector, ((0, npad - n), (0, 0)))
    pos_pad = jnp.pad(pos, ((0, npad - n), (0, 0)))
    posx = pos_pad[:, 0]
    posy = pos_pad[:, 1]

    w1a = W_e1[:, :sd].T
    w1b = W_e1[:, sd:2 * sd].T
    w1c = W_e1[:, 2 * sd].reshape(1, sd)
    w2t = W_e2.T
    wn1a = W_n1[:, :sd].T
    wn1b = W_n1[:, sd:].T
    wn2 = W_n2.T
    be1 = b_e1.reshape(1, sd)
    be2 = b_e2.reshape(1, sd)
    bn1 = b_n1.reshape(1, sd)
    bn2 = b_n2.reshape(1, sd)
    bc = b_c.reshape(1, 1)
    bg = b_g.reshape(1, 1)

    rows = jnp.arange(_BE, dtype=jnp.int32)
    lanes = jnp.arange(128, dtype=jnp.int32)
    lmask = (lanes[None, :] == (rows[:, None] % 128)).astype(jnp.float32)
    chk = jnp.arange(_BE // 128, dtype=jnp.int32)
    sel = ((rows[:, None] // 128) == chk[None, :]).astype(jnp.float32)
    selt = sel.T

    ts, td = _pre_call(h_pad, w1a, w1b, npad)
    zeros_hbm = jnp.zeros((npad, 128), jnp.float32)
    seg_chunks = n_chunks // _NSEG
    seg_epad = seg_chunks * _CHUNK
    aggs, avs = [], []
    for g in range(_NSEG):
        sl = slice(g * seg_chunks, (g + 1) * seg_chunks)
        gsum, dx2d, dy2d = _gather_call(ts, td, posx, posy, src2d[sl],
                                        dst2d[sl], seg_chunks, npad)
        mout, wxy2d = _edge_call(gsum, dx2d, dy2d, lmask, sel, selt,
                                 w2t.astype(jnp.bfloat16), w1c, be1, be2,
                                 W_c, bc, seg_epad)
        agg, av = _scatter_call(dst2d[sl], mout, wxy2d, zeros_hbm,
                                seg_chunks, npad)
        aggs.append(agg)
        avs.append(av)
    oh, ov = _node_call(h_pad, hv_pad, aggs, avs, wn1a, wn1b, bn1,
                        wn2, bn2, W_g, bg, npad)
    return (oh[:n], ov[:n])
